# baseline (speedup 1.0000x reference)
import functools

import jax
import jax.numpy as jnp
from jax import lax
from jax.experimental import pallas as pl
from jax.experimental.pallas import tpu as pltpu
from jax.experimental.pallas import tpu_sc as plsc

_N = 10000
_E = 320000
_G = 128
_HID = 128
_EMB = 64
_IN_CH = 8

_NC = 2
_NS = 16
_NW = _NC * _NS
_CHUNK = 80
_NBUF = 3
_EPW = _E // _NW
_NCH = _EPW // _CHUNK
_MAIN = _NCH - (_NCH % _NBUF)
_NROWS = _N
_RPT = 1000
_NZT = _N // _RPT

_BN = 2000
_NB = _N // _BN


def _make_edge_agg(C):
  mesh = plsc.VectorSubcoreMesh(core_axis_name="c", subcore_axis_name="s")

  @functools.partial(
      pl.kernel,
      out_type=jax.ShapeDtypeStruct((_NC, _N, C), jnp.float32),
      mesh=mesh,
      scratch_types=[
          pltpu.VMEM((_NCH, _CHUNK), jnp.int32),
          pltpu.VMEM((_NCH, _CHUNK), jnp.int32),
          pltpu.VMEM((_NBUF, _CHUNK, C), jnp.float32),
          pltpu.VMEM_SHARED((_NROWS, C), jnp.float32),
          pltpu.SemaphoreType.DMA,
          pltpu.SemaphoreType.DMA,
          pltpu.SemaphoreType.DMA,
      ],
      compiler_params=pltpu.CompilerParams(use_tc_tiling_on_sc=False),
  )
  def agg(h_hbm, src_hbm, dst_hbm, zeros_hbm, out_hbm,
          idx_s, idx_d, rows, acc_sh, sem0, sem1, sem2):
    sems = (sem0, sem1, sem2)
    c = lax.axis_index("c")
    s = lax.axis_index("s")
    @pl.when(s < _NZT)
    def _():
      pltpu.sync_copy(zeros_hbm.at[pl.ds(s * _RPT, _RPT)],
                      acc_sh.at[pl.ds(s * _RPT, _RPT)])
    pltpu.sync_copy(src_hbm.at[c, s], idx_s)
    pltpu.sync_copy(dst_hbm.at[c, s], idx_d)
    plsc.subcore_barrier()

    def start_gather(j, b):
      pltpu.make_async_copy(h_hbm.at[idx_s.at[j]], rows.at[b], sems[b]).start()

    def wait_gather(j, b):
      pltpu.make_async_copy(h_hbm.at[idx_s.at[j]], rows.at[b], sems[b]).wait()

    def start_scatter(j, b):
      pltpu.async_copy(rows.at[b], acc_sh.at[idx_d.at[j]], sems[b], add=True)

    def wait_scatter(j, b):
      pltpu.make_async_copy(rows.at[b], acc_sh.at[idx_d.at[j]], sems[b]).wait()

    for b in range(_NBUF):
      start_gather(b, b)

    @pl.loop(0, _MAIN, step=_NBUF)
    def _(jj):
      for b in range(_NBUF):
        j = jj + b
        wait_gather(j, b)
        start_scatter(j, b)
        bn = (b + 1) % _NBUF

        @pl.when((jj >= 2 - b) & (jj < _NCH - 1 - b))
        def _():
          wait_scatter(j - 2, bn)
          start_gather(j + 1, bn)

    for j in range(_MAIN, _NCH):
      b = j % _NBUF
      if j > _MAIN:
        wait_scatter(j - _NBUF, b)
        start_gather(j, b)
      wait_gather(j, b)
      start_scatter(j, b)
    drain_lo = _NCH - _NBUF if _MAIN < _NCH else _NCH - (_NBUF - 1)
    for j in range(drain_lo, _NCH):
      wait_scatter(j, j % _NBUF)

    plsc.subcore_barrier()
    @pl.when(s < _NZT)
    def _():
      pltpu.sync_copy(acc_sh.at[pl.ds(s * _RPT, _RPT)],
                      out_hbm.at[c, pl.ds(s * _RPT, _RPT)])

  return agg


_agg128 = _make_edge_agg(_HID)


def _emb_body(x_ref, we_ref, w1_ref, o_ref):
  wc = jax.lax.dot(we_ref[...], w1_ref[...],
                   preferred_element_type=jnp.float32)
  o_ref[...] = jax.lax.dot(x_ref[...], wc,
                           preferred_element_type=jnp.float32)


def _emb(x, W_emb, W1):
  return pl.pallas_call(
      _emb_body,
      grid=(_NB,),
      in_specs=[
          pl.BlockSpec((_BN, 57), lambda i: (i, 0)),
          pl.BlockSpec((57, _IN_CH), lambda i: (0, 0)),
          pl.BlockSpec((_IN_CH, _HID), lambda i: (0, 0)),
      ],
      out_specs=pl.BlockSpec((_BN, _HID), lambda i: (i, 0)),
      out_shape=jax.ShapeDtypeStruct((_N, _HID), jnp.float32),
  )(x, W_emb, W1)


def _bias_relu_body(p_ref, b_ref, o_ref):
  o_ref[...] = jnp.maximum(p_ref[0] + p_ref[1] + b_ref[...], 0.0)


def _bias_relu(p, b):
  return pl.pallas_call(
      _bias_relu_body,
      grid=(_NB,),
      in_specs=[
          pl.BlockSpec((_NC, _BN, _HID), lambda i: (0, i, 0)),
          pl.BlockSpec((1, _HID), lambda i: (0, 0)),
      ],
      out_specs=pl.BlockSpec((_BN, _HID), lambda i: (i, 0)),
      out_shape=jax.ShapeDtypeStruct((_N, _HID), jnp.float32),
  )(p, b.reshape(1, _HID))


def _mm_body(p_ref, w_ref, b_ref, o_ref):
  z = jax.lax.dot(p_ref[0] + p_ref[1], w_ref[...],
                  preferred_element_type=jnp.float32) + b_ref[...]
  o_ref[...] = jnp.maximum(z, 0.0)


def _mm_res_body(p_ref, w_ref, b_ref, r_ref, o_ref):
  z = jax.lax.dot(p_ref[0] + p_ref[1], w_ref[...],
                  preferred_element_type=jnp.float32) + b_ref[...]
  o_ref[...] = jnp.maximum(z, 0.0) + r_ref[...]


def _mm(p, W, b):
  K = p.shape[-1]
  return pl.pallas_call(
      _mm_body,
      grid=(_NB,),
      in_specs=[
          pl.BlockSpec((_NC, _BN, K), lambda i: (0, i, 0)),
          pl.BlockSpec((K, _HID), lambda i: (0, 0)),
          pl.BlockSpec((1, _HID), lambda i: (0, 0)),
      ],
      out_specs=pl.BlockSpec((_BN, _HID), lambda i: (i, 0)),
      out_shape=jax.ShapeDtypeStruct((_N, _HID), jnp.float32),
  )(p, W, b.reshape(1, _HID))


def _mm_res(p, W, b, res):
  return pl.pallas_call(
      _mm_res_body,
      grid=(_NB,),
      in_specs=[
          pl.BlockSpec((_NC, _BN, _HID), lambda i: (0, i, 0)),
          pl.BlockSpec((_HID, _HID), lambda i: (0, 0)),
          pl.BlockSpec((1, _HID), lambda i: (0, 0)),
          pl.BlockSpec((_BN, _HID), lambda i: (i, 0)),
      ],
      out_specs=pl.BlockSpec((_BN, _HID), lambda i: (i, 0)),
      out_shape=jax.ShapeDtypeStruct((_N, _HID), jnp.float32),
  )(p, W, b.reshape(1, _HID), res)


def _final_body(p_ref, w_ref, b_ref, r_ref, seg_ref, wp1_ref, wp2_ref,
                bp2_ref, o_ref, acc_ref):
  i = pl.program_id(0)
  h3 = jnp.maximum(
      jax.lax.dot(p_ref[0] + p_ref[1], w_ref[...],
                  preferred_element_type=jnp.float32) + b_ref[...],
      0.0) + r_ref[...]
  seg = seg_ref[0, 0]
  onehot = (seg[:, None] ==
            lax.broadcasted_iota(jnp.int32, (_BN, _G), 1)).astype(jnp.float32)
  contrib = jax.lax.dot_general(onehot, h3, (((0,), (0,)), ((), ())),
                                preferred_element_type=jnp.float32)

  @pl.when(i == 0)
  def _():
    acc_ref[...] = jnp.zeros_like(acc_ref)

  acc_ref[...] += contrib

  @pl.when(i == _NB - 1)
  def _():
    ge = acc_ref[...]
    t = jnp.maximum(jax.lax.dot(ge, wp1_ref[...],
                                preferred_element_type=jnp.float32), 0.0)
    o_ref[...] = jax.lax.dot(t, wp2_ref[...],
                             preferred_element_type=jnp.float32) + bp2_ref[...]


def _final(p, W3, b3, res, seg3d, Wp1, Wp2, bp2):
  return pl.pallas_call(
      _final_body,
      grid=(_NB,),
      in_specs=[
          pl.BlockSpec((_NC, _BN, _HID), lambda i: (0, i, 0)),
          pl.BlockSpec((_HID, _HID), lambda i: (0, 0)),
          pl.BlockSpec((1, _HID), lambda i: (0, 0)),
          pl.BlockSpec((_BN, _HID), lambda i: (i, 0)),
          pl.BlockSpec((1, 1, _BN), lambda i: (i, 0, 0)),
          pl.BlockSpec((_HID, _EMB), lambda i: (0, 0)),
          pl.BlockSpec((_EMB, 1), lambda i: (0, 0)),
          pl.BlockSpec((1, 1), lambda i: (0, 0)),
      ],
      out_specs=pl.BlockSpec((_G, 1), lambda i: (0, 0)),
      out_shape=jax.ShapeDtypeStruct((_G, 1), jnp.float32),
      scratch_shapes=[pltpu.VMEM((_G, _HID), jnp.float32)],
  )(p, W3, b3.reshape(1, _HID), res, seg3d, Wp1, Wp2, bp2.reshape(1, 1))


def kernel(x, edge_index, segment_ids, W_emb, W1, b1, W2, b2, W3, b3,
           Wp1, Wp2, bp2):
  src = edge_index[0].reshape(_NC, _NS, _NCH, _CHUNK)
  dst = edge_index[1].reshape(_NC, _NS, _NCH, _CHUNK)
  z128 = jnp.zeros((_N, _HID), jnp.float32)
  seg3d = segment_ids.reshape(_NB, 1, _BN)

  g0 = _emb(x, W_emb, W1)
  p1 = _agg128(g0, src, dst, z128)
  h1 = _bias_relu(p1, b1)
  p2 = _agg128(h1, src, dst, z128)
  h2 = _mm_res(p2, W2, b2, h1)
  p3 = _agg128(h2, src, dst, z128)
  pred = _final(p3, W3, b3, h2, seg3d, Wp1, Wp2, bp2)
  return pred


# speedup vs baseline: 1.4956x; 1.4956x over previous
import functools

import jax
import jax.numpy as jnp
from jax import lax
from jax.experimental import pallas as pl
from jax.experimental.pallas import tpu as pltpu
from jax.experimental.pallas import tpu_sc as plsc

_N = 10000
_E = 320000
_G = 128
_HID = 128
_EMB = 64
_IN_CH = 8

_NC = 2
_NS = 16
_NW = _NC * _NS
_CHUNK = 80
_NBUF = 3
_EPW = _E // _NW
_NCH = _EPW // _CHUNK
_MAIN = _NCH - (_NCH % _NBUF)
_NROWS = _N
_RPT = 1000
_NZT = _N // _RPT

_BN = 2000
_NB = _N // _BN


def _make_edge_agg(C):
  mesh = plsc.VectorSubcoreMesh(core_axis_name="c", subcore_axis_name="s")

  @functools.partial(
      pl.kernel,
      out_type=jax.ShapeDtypeStruct((_NC, _N, C), jnp.float32),
      mesh=mesh,
      scratch_types=[
          pltpu.VMEM((_NCH, _CHUNK), jnp.int32),
          pltpu.VMEM((_NCH, _CHUNK), jnp.int32),
          pltpu.VMEM((_NBUF, _CHUNK, C), jnp.float32),
          pltpu.VMEM_SHARED((_NROWS, C), jnp.float32),
          pltpu.SemaphoreType.DMA,
          pltpu.SemaphoreType.DMA,
          pltpu.SemaphoreType.DMA,
      ],
      compiler_params=pltpu.CompilerParams(use_tc_tiling_on_sc=False),
  )
  def agg(h_hbm, src_hbm, dst_hbm, zeros_hbm, out_hbm,
          idx_s, idx_d, rows, acc_sh, sem0, sem1, sem2):
    sems = (sem0, sem1, sem2)
    c = lax.axis_index("c")
    s = lax.axis_index("s")
    @pl.when(s < _NZT)
    def _():
      pltpu.sync_copy(zeros_hbm.at[pl.ds(s * _RPT, _RPT)],
                      acc_sh.at[pl.ds(s * _RPT, _RPT)])
    pltpu.sync_copy(src_hbm.at[c, s], idx_s)
    pltpu.sync_copy(dst_hbm.at[c, s], idx_d)
    plsc.subcore_barrier()

    def start_gather(j, b):
      pltpu.make_async_copy(h_hbm.at[idx_s.at[j]], rows.at[b], sems[b]).start()

    def wait_gather(j, b):
      pltpu.make_async_copy(h_hbm.at[idx_s.at[j]], rows.at[b], sems[b]).wait()

    for b in range(_NBUF):
      start_gather(b, b)

    @pl.loop(0, _MAIN, step=_NBUF)
    def _(jj):
      for b in range(_NBUF):
        j = jj + b
        wait_gather(j, b)
        pltpu.sync_copy(rows.at[b], acc_sh.at[idx_d.at[j]], add=True)

        @pl.when(jj < _NCH - _NBUF - b)
        def _():
          start_gather(j + _NBUF, b)

    for j in range(_MAIN, _NCH):
      wait_gather(j, j % _NBUF)
      pltpu.sync_copy(rows.at[j % _NBUF], acc_sh.at[idx_d.at[j]], add=True)

    plsc.subcore_barrier()
    @pl.when(s < _NZT)
    def _():
      pltpu.sync_copy(acc_sh.at[pl.ds(s * _RPT, _RPT)],
                      out_hbm.at[c, pl.ds(s * _RPT, _RPT)])

  return agg


_agg128 = _make_edge_agg(_HID)


def _emb_body(x_ref, we_ref, w1_ref, o_ref):
  wc = jax.lax.dot(we_ref[...], w1_ref[...],
                   preferred_element_type=jnp.float32)
  o_ref[...] = jax.lax.dot(x_ref[...], wc,
                           preferred_element_type=jnp.float32)


def _emb(x, W_emb, W1):
  return pl.pallas_call(
      _emb_body,
      grid=(_NB,),
      in_specs=[
          pl.BlockSpec((_BN, 57), lambda i: (i, 0)),
          pl.BlockSpec((57, _IN_CH), lambda i: (0, 0)),
          pl.BlockSpec((_IN_CH, _HID), lambda i: (0, 0)),
      ],
      out_specs=pl.BlockSpec((_BN, _HID), lambda i: (i, 0)),
      out_shape=jax.ShapeDtypeStruct((_N, _HID), jnp.float32),
  )(x, W_emb, W1)


def _bias_relu_body(p_ref, b_ref, o_ref):
  o_ref[...] = jnp.maximum(p_ref[0] + p_ref[1] + b_ref[...], 0.0)


def _bias_relu(p, b):
  return pl.pallas_call(
      _bias_relu_body,
      grid=(_NB,),
      in_specs=[
          pl.BlockSpec((_NC, _BN, _HID), lambda i: (0, i, 0)),
          pl.BlockSpec((1, _HID), lambda i: (0, 0)),
      ],
      out_specs=pl.BlockSpec((_BN, _HID), lambda i: (i, 0)),
      out_shape=jax.ShapeDtypeStruct((_N, _HID), jnp.float32),
  )(p, b.reshape(1, _HID))


def _mm_body(p_ref, w_ref, b_ref, o_ref):
  z = jax.lax.dot(p_ref[0] + p_ref[1], w_ref[...],
                  preferred_element_type=jnp.float32) + b_ref[...]
  o_ref[...] = jnp.maximum(z, 0.0)


def _mm_res_body(p_ref, w_ref, b_ref, r_ref, o_ref):
  z = jax.lax.dot(p_ref[0] + p_ref[1], w_ref[...],
                  preferred_element_type=jnp.float32) + b_ref[...]
  o_ref[...] = jnp.maximum(z, 0.0) + r_ref[...]


def _mm(p, W, b):
  K = p.shape[-1]
  return pl.pallas_call(
      _mm_body,
      grid=(_NB,),
      in_specs=[
          pl.BlockSpec((_NC, _BN, K), lambda i: (0, i, 0)),
          pl.BlockSpec((K, _HID), lambda i: (0, 0)),
          pl.BlockSpec((1, _HID), lambda i: (0, 0)),
      ],
      out_specs=pl.BlockSpec((_BN, _HID), lambda i: (i, 0)),
      out_shape=jax.ShapeDtypeStruct((_N, _HID), jnp.float32),
  )(p, W, b.reshape(1, _HID))


def _mm_res(p, W, b, res):
  return pl.pallas_call(
      _mm_res_body,
      grid=(_NB,),
      in_specs=[
          pl.BlockSpec((_NC, _BN, _HID), lambda i: (0, i, 0)),
          pl.BlockSpec((_HID, _HID), lambda i: (0, 0)),
          pl.BlockSpec((1, _HID), lambda i: (0, 0)),
          pl.BlockSpec((_BN, _HID), lambda i: (i, 0)),
      ],
      out_specs=pl.BlockSpec((_BN, _HID), lambda i: (i, 0)),
      out_shape=jax.ShapeDtypeStruct((_N, _HID), jnp.float32),
  )(p, W, b.reshape(1, _HID), res)


def _final_body(p_ref, w_ref, b_ref, r_ref, seg_ref, wp1_ref, wp2_ref,
                bp2_ref, o_ref, acc_ref):
  i = pl.program_id(0)
  h3 = jnp.maximum(
      jax.lax.dot(p_ref[0] + p_ref[1], w_ref[...],
                  preferred_element_type=jnp.float32) + b_ref[...],
      0.0) + r_ref[...]
  seg = seg_ref[0, 0]
  onehot = (seg[:, None] ==
            lax.broadcasted_iota(jnp.int32, (_BN, _G), 1)).astype(jnp.float32)
  contrib = jax.lax.dot_general(onehot, h3, (((0,), (0,)), ((), ())),
                                preferred_element_type=jnp.float32)

  @pl.when(i == 0)
  def _():
    acc_ref[...] = jnp.zeros_like(acc_ref)

  acc_ref[...] += contrib

  @pl.when(i == _NB - 1)
  def _():
    ge = acc_ref[...]
    t = jnp.maximum(jax.lax.dot(ge, wp1_ref[...],
                                preferred_element_type=jnp.float32), 0.0)
    o_ref[...] = jax.lax.dot(t, wp2_ref[...],
                             preferred_element_type=jnp.float32) + bp2_ref[...]


def _final(p, W3, b3, res, seg3d, Wp1, Wp2, bp2):
  return pl.pallas_call(
      _final_body,
      grid=(_NB,),
      in_specs=[
          pl.BlockSpec((_NC, _BN, _HID), lambda i: (0, i, 0)),
          pl.BlockSpec((_HID, _HID), lambda i: (0, 0)),
          pl.BlockSpec((1, _HID), lambda i: (0, 0)),
          pl.BlockSpec((_BN, _HID), lambda i: (i, 0)),
          pl.BlockSpec((1, 1, _BN), lambda i: (i, 0, 0)),
          pl.BlockSpec((_HID, _EMB), lambda i: (0, 0)),
          pl.BlockSpec((_EMB, 1), lambda i: (0, 0)),
          pl.BlockSpec((1, 1), lambda i: (0, 0)),
      ],
      out_specs=pl.BlockSpec((_G, 1), lambda i: (0, 0)),
      out_shape=jax.ShapeDtypeStruct((_G, 1), jnp.float32),
      scratch_shapes=[pltpu.VMEM((_G, _HID), jnp.float32)],
  )(p, W3, b3.reshape(1, _HID), res, seg3d, Wp1, Wp2, bp2.reshape(1, 1))


def kernel(x, edge_index, segment_ids, W_emb, W1, b1, W2, b2, W3, b3,
           Wp1, Wp2, bp2):
  src = edge_index[0].reshape(_NC, _NS, _NCH, _CHUNK)
  dst = edge_index[1].reshape(_NC, _NS, _NCH, _CHUNK)
  z128 = jnp.zeros((_N, _HID), jnp.float32)
  seg3d = segment_ids.reshape(_NB, 1, _BN)

  g0 = _emb(x, W_emb, W1)
  p1 = _agg128(g0, src, dst, z128)
  h1 = _bias_relu(p1, b1)
  p2 = _agg128(h1, src, dst, z128)
  h2 = _mm_res(p2, W2, b2, h1)
  p3 = _agg128(h2, src, dst, z128)
  pred = _final(p3, W3, b3, h2, seg3d, Wp1, Wp2, bp2)
  return pred


# speedup vs baseline: 1.5343x; 1.0258x over previous
import functools

import jax
import jax.numpy as jnp
from jax import lax
from jax.experimental import pallas as pl
from jax.experimental.pallas import tpu as pltpu
from jax.experimental.pallas import tpu_sc as plsc

_N = 10000
_E = 320000
_G = 128
_HID = 128
_EMB = 64
_IN_CH = 8

_NC = 2
_NS = 16
_NW = _NC * _NS
_CHUNK = 40
_NBUF = 5
_EPW = _E // _NW
_NCH = _EPW // _CHUNK
_MAIN = _NCH - (_NCH % _NBUF)
_NROWS = _N
_RPT = 1000
_NZT = _N // _RPT

_BN = 2000
_NB = _N // _BN


def _make_edge_agg(C):
  mesh = plsc.VectorSubcoreMesh(core_axis_name="c", subcore_axis_name="s")

  @functools.partial(
      pl.kernel,
      out_type=jax.ShapeDtypeStruct((_NC, _N, C), jnp.float32),
      mesh=mesh,
      scratch_types=[
          pltpu.VMEM((_NCH, _CHUNK), jnp.int32),
          pltpu.VMEM((_NCH, _CHUNK), jnp.int32),
          pltpu.VMEM((_NBUF, _CHUNK, C), jnp.float32),
          pltpu.VMEM_SHARED((_NROWS, C), jnp.float32),
          pltpu.SemaphoreType.DMA,
          pltpu.SemaphoreType.DMA,
          pltpu.SemaphoreType.DMA,
          pltpu.SemaphoreType.DMA,
          pltpu.SemaphoreType.DMA,
      ],
      compiler_params=pltpu.CompilerParams(use_tc_tiling_on_sc=False),
  )
  def agg(h_hbm, src_hbm, dst_hbm, zeros_hbm, out_hbm,
          idx_s, idx_d, rows, acc_sh, sem0, sem1, sem2, sem3, sem4):
    sems = (sem0, sem1, sem2, sem3, sem4)
    c = lax.axis_index("c")
    s = lax.axis_index("s")
    @pl.when(s < _NZT)
    def _():
      pltpu.sync_copy(zeros_hbm.at[pl.ds(s * _RPT, _RPT)],
                      acc_sh.at[pl.ds(s * _RPT, _RPT)])
    pltpu.sync_copy(src_hbm.at[c, s], idx_s)
    pltpu.sync_copy(dst_hbm.at[c, s], idx_d)
    plsc.subcore_barrier()

    def start_gather(j, b):
      pltpu.make_async_copy(h_hbm.at[idx_s.at[j]], rows.at[b], sems[b]).start()

    def wait_gather(j, b):
      pltpu.make_async_copy(h_hbm.at[idx_s.at[j]], rows.at[b], sems[b]).wait()

    for b in range(_NBUF):
      start_gather(b, b)

    @pl.loop(0, _MAIN, step=_NBUF)
    def _(jj):
      for b in range(_NBUF):
        j = jj + b
        wait_gather(j, b)
        pltpu.sync_copy(rows.at[b], acc_sh.at[idx_d.at[j]], add=True)

        @pl.when(jj < _NCH - _NBUF - b)
        def _():
          start_gather(j + _NBUF, b)

    for j in range(_MAIN, _NCH):
      wait_gather(j, j % _NBUF)
      pltpu.sync_copy(rows.at[j % _NBUF], acc_sh.at[idx_d.at[j]], add=True)

    plsc.subcore_barrier()
    @pl.when(s < _NZT)
    def _():
      pltpu.sync_copy(acc_sh.at[pl.ds(s * _RPT, _RPT)],
                      out_hbm.at[c, pl.ds(s * _RPT, _RPT)])

  return agg


_agg128 = _make_edge_agg(_HID)


def _emb_body(x_ref, we_ref, w1_ref, o_ref):
  wc = jax.lax.dot(we_ref[...], w1_ref[...],
                   preferred_element_type=jnp.float32)
  o_ref[...] = jax.lax.dot(x_ref[...], wc,
                           preferred_element_type=jnp.float32)


def _emb(x, W_emb, W1):
  return pl.pallas_call(
      _emb_body,
      grid=(_NB,),
      in_specs=[
          pl.BlockSpec((_BN, 57), lambda i: (i, 0)),
          pl.BlockSpec((57, _IN_CH), lambda i: (0, 0)),
          pl.BlockSpec((_IN_CH, _HID), lambda i: (0, 0)),
      ],
      out_specs=pl.BlockSpec((_BN, _HID), lambda i: (i, 0)),
      out_shape=jax.ShapeDtypeStruct((_N, _HID), jnp.float32),
  )(x, W_emb, W1)


def _bias_relu_body(p_ref, b_ref, o_ref):
  o_ref[...] = jnp.maximum(p_ref[0] + p_ref[1] + b_ref[...], 0.0)


def _bias_relu(p, b):
  return pl.pallas_call(
      _bias_relu_body,
      grid=(_NB,),
      in_specs=[
          pl.BlockSpec((_NC, _BN, _HID), lambda i: (0, i, 0)),
          pl.BlockSpec((1, _HID), lambda i: (0, 0)),
      ],
      out_specs=pl.BlockSpec((_BN, _HID), lambda i: (i, 0)),
      out_shape=jax.ShapeDtypeStruct((_N, _HID), jnp.float32),
  )(p, b.reshape(1, _HID))


def _mm_body(p_ref, w_ref, b_ref, o_ref):
  z = jax.lax.dot(p_ref[0] + p_ref[1], w_ref[...],
                  preferred_element_type=jnp.float32) + b_ref[...]
  o_ref[...] = jnp.maximum(z, 0.0)


def _mm_res_body(p_ref, w_ref, b_ref, r_ref, o_ref):
  z = jax.lax.dot(p_ref[0] + p_ref[1], w_ref[...],
                  preferred_element_type=jnp.float32) + b_ref[...]
  o_ref[...] = jnp.maximum(z, 0.0) + r_ref[...]


def _mm(p, W, b):
  K = p.shape[-1]
  return pl.pallas_call(
      _mm_body,
      grid=(_NB,),
      in_specs=[
          pl.BlockSpec((_NC, _BN, K), lambda i: (0, i, 0)),
          pl.BlockSpec((K, _HID), lambda i: (0, 0)),
          pl.BlockSpec((1, _HID), lambda i: (0, 0)),
      ],
      out_specs=pl.BlockSpec((_BN, _HID), lambda i: (i, 0)),
      out_shape=jax.ShapeDtypeStruct((_N, _HID), jnp.float32),
  )(p, W, b.reshape(1, _HID))


def _mm_res(p, W, b, res):
  return pl.pallas_call(
      _mm_res_body,
      grid=(_NB,),
      in_specs=[
          pl.BlockSpec((_NC, _BN, _HID), lambda i: (0, i, 0)),
          pl.BlockSpec((_HID, _HID), lambda i: (0, 0)),
          pl.BlockSpec((1, _HID), lambda i: (0, 0)),
          pl.BlockSpec((_BN, _HID), lambda i: (i, 0)),
      ],
      out_specs=pl.BlockSpec((_BN, _HID), lambda i: (i, 0)),
      out_shape=jax.ShapeDtypeStruct((_N, _HID), jnp.float32),
  )(p, W, b.reshape(1, _HID), res)


def _final_body(p_ref, w_ref, b_ref, r_ref, seg_ref, wp1_ref, wp2_ref,
                bp2_ref, o_ref, acc_ref):
  i = pl.program_id(0)
  h3 = jnp.maximum(
      jax.lax.dot(p_ref[0] + p_ref[1], w_ref[...],
                  preferred_element_type=jnp.float32) + b_ref[...],
      0.0) + r_ref[...]
  seg = seg_ref[0, 0]
  onehot = (seg[:, None] ==
            lax.broadcasted_iota(jnp.int32, (_BN, _G), 1)).astype(jnp.float32)
  contrib = jax.lax.dot_general(onehot, h3, (((0,), (0,)), ((), ())),
                                preferred_element_type=jnp.float32)

  @pl.when(i == 0)
  def _():
    acc_ref[...] = jnp.zeros_like(acc_ref)

  acc_ref[...] += contrib

  @pl.when(i == _NB - 1)
  def _():
    ge = acc_ref[...]
    t = jnp.maximum(jax.lax.dot(ge, wp1_ref[...],
                                preferred_element_type=jnp.float32), 0.0)
    o_ref[...] = jax.lax.dot(t, wp2_ref[...],
                             preferred_element_type=jnp.float32) + bp2_ref[...]


def _final(p, W3, b3, res, seg3d, Wp1, Wp2, bp2):
  return pl.pallas_call(
      _final_body,
      grid=(_NB,),
      in_specs=[
          pl.BlockSpec((_NC, _BN, _HID), lambda i: (0, i, 0)),
          pl.BlockSpec((_HID, _HID), lambda i: (0, 0)),
          pl.BlockSpec((1, _HID), lambda i: (0, 0)),
          pl.BlockSpec((_BN, _HID), lambda i: (i, 0)),
          pl.BlockSpec((1, 1, _BN), lambda i: (i, 0, 0)),
          pl.BlockSpec((_HID, _EMB), lambda i: (0, 0)),
          pl.BlockSpec((_EMB, 1), lambda i: (0, 0)),
          pl.BlockSpec((1, 1), lambda i: (0, 0)),
      ],
      out_specs=pl.BlockSpec((_G, 1), lambda i: (0, 0)),
      out_shape=jax.ShapeDtypeStruct((_G, 1), jnp.float32),
      scratch_shapes=[pltpu.VMEM((_G, _HID), jnp.float32)],
  )(p, W3, b3.reshape(1, _HID), res, seg3d, Wp1, Wp2, bp2.reshape(1, 1))


def kernel(x, edge_index, segment_ids, W_emb, W1, b1, W2, b2, W3, b3,
           Wp1, Wp2, bp2):
  src = edge_index[0].reshape(_NC, _NS, _NCH, _CHUNK)
  dst = edge_index[1].reshape(_NC, _NS, _NCH, _CHUNK)
  z128 = jnp.zeros((_N, _HID), jnp.float32)
  seg3d = segment_ids.reshape(_NB, 1, _BN)

  g0 = _emb(x, W_emb, W1)
  p1 = _agg128(g0, src, dst, z128)
  h1 = _bias_relu(p1, b1)
  p2 = _agg128(h1, src, dst, z128)
  h2 = _mm_res(p2, W2, b2, h1)
  p3 = _agg128(h2, src, dst, z128)
  pred = _final(p3, W3, b3, h2, seg3d, Wp1, Wp2, bp2)
  return pred


# speedup vs baseline: 1.5423x; 1.0052x over previous
import functools

import jax
import jax.numpy as jnp
from jax import lax
from jax.experimental import pallas as pl
from jax.experimental.pallas import tpu as pltpu
from jax.experimental.pallas import tpu_sc as plsc

_N = 10000
_E = 320000
_G = 128
_HID = 128
_EMB = 64
_IN_CH = 8

_NC = 2
_NS = 16
_NW = _NC * _NS
_CHUNK = 40
_NBUF = 6
_EPW = _E // _NW
_NCH = _EPW // _CHUNK
_MAIN = _NCH - (_NCH % _NBUF)
_NROWS = _N
_RPT = 1000
_NZT = _N // _RPT

_BN = 2000
_NB = _N // _BN


def _make_edge_agg(C):
  mesh = plsc.VectorSubcoreMesh(core_axis_name="c", subcore_axis_name="s")

  @functools.partial(
      pl.kernel,
      out_type=jax.ShapeDtypeStruct((_NC, _N, C), jnp.float32),
      mesh=mesh,
      scratch_types=[
          pltpu.VMEM((_NCH, _CHUNK), jnp.int32),
          pltpu.VMEM((_NCH, _CHUNK), jnp.int32),
          pltpu.VMEM((_NBUF, _CHUNK, C), jnp.float32),
          pltpu.VMEM_SHARED((_NROWS, C), jnp.float32),
          pltpu.SemaphoreType.DMA,
          pltpu.SemaphoreType.DMA,
          pltpu.SemaphoreType.DMA,
          pltpu.SemaphoreType.DMA,
          pltpu.SemaphoreType.DMA,
          pltpu.SemaphoreType.DMA,
      ],
      compiler_params=pltpu.CompilerParams(use_tc_tiling_on_sc=False),
  )
  def agg(h_hbm, src_hbm, dst_hbm, zeros_hbm, out_hbm,
          idx_s, idx_d, rows, acc_sh, sem0, sem1, sem2, sem3, sem4, sem5):
    sems = (sem0, sem1, sem2, sem3, sem4, sem5)
    c = lax.axis_index("c")
    s = lax.axis_index("s")
    @pl.when(s < _NZT)
    def _():
      pltpu.sync_copy(zeros_hbm.at[pl.ds(s * _RPT, _RPT)],
                      acc_sh.at[pl.ds(s * _RPT, _RPT)])
    pltpu.sync_copy(src_hbm.at[c, s], idx_s)
    pltpu.sync_copy(dst_hbm.at[c, s], idx_d)
    plsc.subcore_barrier()

    def start_gather(j, b):
      pltpu.make_async_copy(h_hbm.at[idx_s.at[j]], rows.at[b], sems[b]).start()

    def wait_gather(j, b):
      pltpu.make_async_copy(h_hbm.at[idx_s.at[j]], rows.at[b], sems[b]).wait()

    for b in range(_NBUF):
      start_gather(b, b)

    @pl.loop(0, _MAIN, step=_NBUF)
    def _(jj):
      for b in range(_NBUF):
        j = jj + b
        wait_gather(j, b)
        pltpu.sync_copy(rows.at[b], acc_sh.at[idx_d.at[j]], add=True)

        @pl.when(jj < _NCH - _NBUF - b)
        def _():
          start_gather(j + _NBUF, b)

    for j in range(_MAIN, _NCH):
      wait_gather(j, j % _NBUF)
      pltpu.sync_copy(rows.at[j % _NBUF], acc_sh.at[idx_d.at[j]], add=True)

    plsc.subcore_barrier()
    @pl.when(s < _NZT)
    def _():
      pltpu.sync_copy(acc_sh.at[pl.ds(s * _RPT, _RPT)],
                      out_hbm.at[c, pl.ds(s * _RPT, _RPT)])

  return agg


_agg128 = _make_edge_agg(_HID)


def _emb_body(x_ref, we_ref, w1_ref, o_ref):
  wc = jax.lax.dot(we_ref[...], w1_ref[...],
                   preferred_element_type=jnp.float32)
  o_ref[...] = jax.lax.dot(x_ref[...], wc,
                           preferred_element_type=jnp.float32)


def _emb(x, W_emb, W1):
  return pl.pallas_call(
      _emb_body,
      grid=(_NB,),
      in_specs=[
          pl.BlockSpec((_BN, 57), lambda i: (i, 0)),
          pl.BlockSpec((57, _IN_CH), lambda i: (0, 0)),
          pl.BlockSpec((_IN_CH, _HID), lambda i: (0, 0)),
      ],
      out_specs=pl.BlockSpec((_BN, _HID), lambda i: (i, 0)),
      out_shape=jax.ShapeDtypeStruct((_N, _HID), jnp.float32),
  )(x, W_emb, W1)


def _bias_relu_body(p_ref, b_ref, o_ref):
  o_ref[...] = jnp.maximum(p_ref[0] + p_ref[1] + b_ref[...], 0.0)


def _bias_relu(p, b):
  return pl.pallas_call(
      _bias_relu_body,
      grid=(_NB,),
      in_specs=[
          pl.BlockSpec((_NC, _BN, _HID), lambda i: (0, i, 0)),
          pl.BlockSpec((1, _HID), lambda i: (0, 0)),
      ],
      out_specs=pl.BlockSpec((_BN, _HID), lambda i: (i, 0)),
      out_shape=jax.ShapeDtypeStruct((_N, _HID), jnp.float32),
  )(p, b.reshape(1, _HID))


def _mm_body(p_ref, w_ref, b_ref, o_ref):
  z = jax.lax.dot(p_ref[0] + p_ref[1], w_ref[...],
                  preferred_element_type=jnp.float32) + b_ref[...]
  o_ref[...] = jnp.maximum(z, 0.0)


def _mm_res_body(p_ref, w_ref, b_ref, r_ref, o_ref):
  z = jax.lax.dot(p_ref[0] + p_ref[1], w_ref[...],
                  preferred_element_type=jnp.float32) + b_ref[...]
  o_ref[...] = jnp.maximum(z, 0.0) + r_ref[...]


def _mm(p, W, b):
  K = p.shape[-1]
  return pl.pallas_call(
      _mm_body,
      grid=(_NB,),
      in_specs=[
          pl.BlockSpec((_NC, _BN, K), lambda i: (0, i, 0)),
          pl.BlockSpec((K, _HID), lambda i: (0, 0)),
          pl.BlockSpec((1, _HID), lambda i: (0, 0)),
      ],
      out_specs=pl.BlockSpec((_BN, _HID), lambda i: (i, 0)),
      out_shape=jax.ShapeDtypeStruct((_N, _HID), jnp.float32),
  )(p, W, b.reshape(1, _HID))


def _mm_res(p, W, b, res):
  return pl.pallas_call(
      _mm_res_body,
      grid=(_NB,),
      in_specs=[
          pl.BlockSpec((_NC, _BN, _HID), lambda i: (0, i, 0)),
          pl.BlockSpec((_HID, _HID), lambda i: (0, 0)),
          pl.BlockSpec((1, _HID), lambda i: (0, 0)),
          pl.BlockSpec((_BN, _HID), lambda i: (i, 0)),
      ],
      out_specs=pl.BlockSpec((_BN, _HID), lambda i: (i, 0)),
      out_shape=jax.ShapeDtypeStruct((_N, _HID), jnp.float32),
  )(p, W, b.reshape(1, _HID), res)


def _final_body(p_ref, w_ref, b_ref, r_ref, seg_ref, wp1_ref, wp2_ref,
                bp2_ref, o_ref, acc_ref):
  i = pl.program_id(0)
  h3 = jnp.maximum(
      jax.lax.dot(p_ref[0] + p_ref[1], w_ref[...],
                  preferred_element_type=jnp.float32) + b_ref[...],
      0.0) + r_ref[...]
  seg = seg_ref[0, 0]
  onehot = (seg[:, None] ==
            lax.broadcasted_iota(jnp.int32, (_BN, _G), 1)).astype(jnp.float32)
  contrib = jax.lax.dot_general(onehot, h3, (((0,), (0,)), ((), ())),
                                preferred_element_type=jnp.float32)

  @pl.when(i == 0)
  def _():
    acc_ref[...] = jnp.zeros_like(acc_ref)

  acc_ref[...] += contrib

  @pl.when(i == _NB - 1)
  def _():
    ge = acc_ref[...]
    t = jnp.maximum(jax.lax.dot(ge, wp1_ref[...],
                                preferred_element_type=jnp.float32), 0.0)
    o_ref[...] = jax.lax.dot(t, wp2_ref[...],
                             preferred_element_type=jnp.float32) + bp2_ref[...]


def _final(p, W3, b3, res, seg3d, Wp1, Wp2, bp2):
  return pl.pallas_call(
      _final_body,
      grid=(_NB,),
      in_specs=[
          pl.BlockSpec((_NC, _BN, _HID), lambda i: (0, i, 0)),
          pl.BlockSpec((_HID, _HID), lambda i: (0, 0)),
          pl.BlockSpec((1, _HID), lambda i: (0, 0)),
          pl.BlockSpec((_BN, _HID), lambda i: (i, 0)),
          pl.BlockSpec((1, 1, _BN), lambda i: (i, 0, 0)),
          pl.BlockSpec((_HID, _EMB), lambda i: (0, 0)),
          pl.BlockSpec((_EMB, 1), lambda i: (0, 0)),
          pl.BlockSpec((1, 1), lambda i: (0, 0)),
      ],
      out_specs=pl.BlockSpec((_G, 1), lambda i: (0, 0)),
      out_shape=jax.ShapeDtypeStruct((_G, 1), jnp.float32),
      scratch_shapes=[pltpu.VMEM((_G, _HID), jnp.float32)],
  )(p, W3, b3.reshape(1, _HID), res, seg3d, Wp1, Wp2, bp2.reshape(1, 1))


def kernel(x, edge_index, segment_ids, W_emb, W1, b1, W2, b2, W3, b3,
           Wp1, Wp2, bp2):
  src = edge_index[0].reshape(_NC, _NS, _NCH, _CHUNK)
  dst = edge_index[1].reshape(_NC, _NS, _NCH, _CHUNK)
  z128 = jnp.zeros((_N, _HID), jnp.float32)
  seg3d = segment_ids.reshape(_NB, 1, _BN)

  g0 = _emb(x, W_emb, W1)
  p1 = _agg128(g0, src, dst, z128)
  h1 = _bias_relu(p1, b1)
  p2 = _agg128(h1, src, dst, z128)
  h2 = _mm_res(p2, W2, b2, h1)
  p3 = _agg128(h2, src, dst, z128)
  pred = _final(p3, W3, b3, h2, seg3d, Wp1, Wp2, bp2)
  return pred
